# baseline (speedup 1.0000x reference)
import jax
import jax.numpy as jnp
from jax.experimental import pallas as pl
from jax.experimental.pallas import tpu as pltpu

_N = 10000
_FEAT = 128
_NHID = 256

_TMC = 200
_NST = _N // _TMC // 2
_NBUF = 2
_TSUP = 2 * _TMC
_NSUP = _N // _TSUP


def _outer(adj_hbm, x_ref, wt_ref, wb_ref, b_ref, w1_ref, b1_ref,
           alpha_ref, w2_ref, b2_ref, out_ref, agg_ref):

    def _mlp_tail(s, orow):
        xm = x_ref[pl.ds(s, _TSUP), :]
        agg = agg_ref[pl.ds(s, _TSUP), :]
        h = jnp.dot(xm, wt_ref[...], preferred_element_type=jnp.float32)
        h += jnp.dot(agg, wb_ref[...], preferred_element_type=jnp.float32)
        h = jnp.maximum(h + b_ref[...], 0.0)
        h1 = jnp.dot(h, w1_ref[...], preferred_element_type=jnp.float32)
        h1 += b1_ref[...]
        h1 = jnp.where(h1 >= 0, h1, alpha_ref[...] * h1)
        pred = jnp.sum(h1 * w2_ref[...], axis=1) + b2_ref[0, 0]
        out_ref[orow, :] = pred

    def _chunk(adj_blk0, adj_blk1):
        i = pl.program_id(0)
        agg_ref[pl.ds(2 * i * _TMC, _TMC), :] = jnp.dot(
            adj_blk0[...], x_ref[...], preferred_element_type=jnp.float32)
        agg_ref[pl.ds((2 * i + 1) * _TMC, _TMC), :] = jnp.dot(
            adj_blk1[...], x_ref[...], preferred_element_type=jnp.float32)
        _mlp_tail(2 * i * _TMC, i)

    pipe = pltpu.emit_pipeline(
        _chunk,
        grid=(_NST,),
        in_specs=[
            pl.BlockSpec((_TMC, _N), lambda i: (2 * i, 0),
                         pipeline_mode=pl.Buffered(buffer_count=_NBUF)),
            pl.BlockSpec((_TMC, _N), lambda i: (2 * i + 1, 0),
                         pipeline_mode=pl.Buffered(buffer_count=_NBUF)),
        ],
    )
    pipe(adj_hbm, adj_hbm)


def kernel(x, adj, W, b, W1, b1, alpha, W2, b2):
    wt = W[:_FEAT]
    wb = W[_FEAT:]
    out = pl.pallas_call(
        _outer,
        grid=(1,),
        in_specs=[
            pl.BlockSpec(memory_space=pltpu.MemorySpace.HBM),
            pl.BlockSpec((_N, _FEAT), lambda i: (0, 0)),
            pl.BlockSpec((_FEAT, _NHID), lambda i: (0, 0)),
            pl.BlockSpec((_FEAT, _NHID), lambda i: (0, 0)),
            pl.BlockSpec((1, _NHID), lambda i: (0, 0)),
            pl.BlockSpec((_NHID, _NHID), lambda i: (0, 0)),
            pl.BlockSpec((1, _NHID), lambda i: (0, 0)),
            pl.BlockSpec((1, _NHID), lambda i: (0, 0)),
            pl.BlockSpec((1, _NHID), lambda i: (0, 0)),
            pl.BlockSpec((1, 1), lambda i: (0, 0)),
        ],
        out_specs=pl.BlockSpec((_NSUP, _TSUP), lambda i: (0, 0)),
        out_shape=jax.ShapeDtypeStruct((_NSUP, _TSUP), jnp.float32),
        scratch_shapes=[pltpu.VMEM((_N, _FEAT), jnp.float32)],
        compiler_params=pltpu.CompilerParams(
            dimension_semantics=("arbitrary",),
        ),
    )(adj, x, wt, wb, b.reshape(1, _NHID), W1, b1.reshape(1, _NHID),
      alpha.reshape(1, _NHID), W2.reshape(1, _NHID), b2.reshape(1, 1))
    return out.reshape(-1)


# speedup vs baseline: 1.0031x; 1.0031x over previous
import jax
import jax.numpy as jnp
from jax.experimental import pallas as pl
from jax.experimental.pallas import tpu as pltpu

_N = 10000
_FEAT = 128
_NHID = 256

_HALF = _N // 2
_TMC = 200
_NCH = _HALF // _TMC
_NBUF = 2
_TSUP = _TMC
_NSUP = _N // _TSUP


def _outer(adj_hbm, x_ref, wt_ref, wb_ref, b_ref, w1_ref, b1_ref,
           alpha_ref, w2_ref, b2_ref, out_ref, agg_ref):

    def _mlp_tail(s, orow):
        xm = x_ref[pl.ds(s, _TSUP), :]
        agg = agg_ref[pl.ds(s, _TSUP), :]
        h = jnp.dot(xm, wt_ref[...], preferred_element_type=jnp.float32)
        h += jnp.dot(agg, wb_ref[...], preferred_element_type=jnp.float32)
        h = jnp.maximum(h + b_ref[...], 0.0)
        h1 = jnp.dot(h, w1_ref[...], preferred_element_type=jnp.float32)
        h1 += b1_ref[...]
        h1 = jnp.where(h1 >= 0, h1, alpha_ref[...] * h1)
        pred = jnp.sum(h1 * w2_ref[...], axis=1) + b2_ref[0, 0]
        out_ref[orow, :] = pred

    def _chunk(adj_blk0, adj_blk1):
        i = pl.program_id(0)
        agg_ref[pl.ds(i * _TMC, _TMC), :] = jnp.dot(
            adj_blk0[...], x_ref[...], preferred_element_type=jnp.float32)
        agg_ref[pl.ds(_HALF + i * _TMC, _TMC), :] = jnp.dot(
            adj_blk1[...], x_ref[...], preferred_element_type=jnp.float32)
        _mlp_tail(i * _TMC, i)
        _mlp_tail(_HALF + i * _TMC, _NCH + i)

    pipe = pltpu.emit_pipeline(
        _chunk,
        grid=(_NCH,),
        in_specs=[
            pl.BlockSpec((_TMC, _N), lambda i: (i, 0),
                         pipeline_mode=pl.Buffered(buffer_count=_NBUF)),
            pl.BlockSpec((_TMC, _N), lambda i: (_NCH + i, 0),
                         pipeline_mode=pl.Buffered(buffer_count=_NBUF)),
        ],
    )
    pipe(adj_hbm, adj_hbm)


def kernel(x, adj, W, b, W1, b1, alpha, W2, b2):
    wt = W[:_FEAT]
    wb = W[_FEAT:]
    out = pl.pallas_call(
        _outer,
        grid=(1,),
        in_specs=[
            pl.BlockSpec(memory_space=pltpu.MemorySpace.HBM),
            pl.BlockSpec((_N, _FEAT), lambda i: (0, 0)),
            pl.BlockSpec((_FEAT, _NHID), lambda i: (0, 0)),
            pl.BlockSpec((_FEAT, _NHID), lambda i: (0, 0)),
            pl.BlockSpec((1, _NHID), lambda i: (0, 0)),
            pl.BlockSpec((_NHID, _NHID), lambda i: (0, 0)),
            pl.BlockSpec((1, _NHID), lambda i: (0, 0)),
            pl.BlockSpec((1, _NHID), lambda i: (0, 0)),
            pl.BlockSpec((1, _NHID), lambda i: (0, 0)),
            pl.BlockSpec((1, 1), lambda i: (0, 0)),
        ],
        out_specs=pl.BlockSpec((_NSUP, _TSUP), lambda i: (0, 0)),
        out_shape=jax.ShapeDtypeStruct((_NSUP, _TSUP), jnp.float32),
        scratch_shapes=[pltpu.VMEM((_N, _FEAT), jnp.float32)],
        compiler_params=pltpu.CompilerParams(
            dimension_semantics=("arbitrary",),
        ),
    )(adj, x, wt, wb, b.reshape(1, _NHID), W1, b1.reshape(1, _NHID),
      alpha.reshape(1, _NHID), W2.reshape(1, _NHID), b2.reshape(1, 1))
    return out.reshape(-1)


# speedup vs baseline: 1.0052x; 1.0021x over previous
import jax
import jax.numpy as jnp
from jax.experimental import pallas as pl
from jax.experimental.pallas import tpu as pltpu

_N = 10000
_FEAT = 128
_NHID = 256

_HALF = _N // 2
_TMC = 200
_NCH = _HALF // _TMC
_NBUF = 2
_TSUP = _TMC
_NSUP = _N // _TSUP


def _outer(adj_hbm, x_ref, wt_ref, wb_ref, b_ref, w1_ref, b1_ref,
           alpha_ref, w2_ref, b2_ref, out_ref, agg_ref):

    def _mlp_tail(s, orow):
        xm = x_ref[pl.ds(s, _TSUP), :]
        agg = agg_ref[pl.ds(s, _TSUP), :]
        h = jnp.dot(xm, wt_ref[...], preferred_element_type=jnp.float32)
        h += jnp.dot(agg, wb_ref[...], preferred_element_type=jnp.float32)
        h = jnp.maximum(h + b_ref[...], 0.0)
        h1 = jnp.dot(h, w1_ref[...], preferred_element_type=jnp.float32)
        h1 += b1_ref[...]
        h1 = jnp.where(h1 >= 0, h1, alpha_ref[...] * h1)
        pred = jnp.sum(h1 * w2_ref[...], axis=1) + b2_ref[0, 0]
        out_ref[orow, :] = pred

    def _chunk(adj_blk0):
        i = pl.program_id(0)
        agg_ref[pl.ds(i * _TMC, _TMC), :] = jnp.dot(
            adj_blk0[...], x_ref[...], preferred_element_type=jnp.float32)
        _mlp_tail(i * _TMC, i)

    pipe = pltpu.emit_pipeline(
        _chunk,
        grid=(2 * _NCH,),
        in_specs=[
            pl.BlockSpec((_TMC, _N), lambda i: (i, 0),
                         pipeline_mode=pl.Buffered(buffer_count=4)),
        ],
    )
    pipe(adj_hbm)


def kernel(x, adj, W, b, W1, b1, alpha, W2, b2):
    wt = W[:_FEAT]
    wb = W[_FEAT:]
    out = pl.pallas_call(
        _outer,
        grid=(1,),
        in_specs=[
            pl.BlockSpec(memory_space=pltpu.MemorySpace.HBM),
            pl.BlockSpec((_N, _FEAT), lambda i: (0, 0)),
            pl.BlockSpec((_FEAT, _NHID), lambda i: (0, 0)),
            pl.BlockSpec((_FEAT, _NHID), lambda i: (0, 0)),
            pl.BlockSpec((1, _NHID), lambda i: (0, 0)),
            pl.BlockSpec((_NHID, _NHID), lambda i: (0, 0)),
            pl.BlockSpec((1, _NHID), lambda i: (0, 0)),
            pl.BlockSpec((1, _NHID), lambda i: (0, 0)),
            pl.BlockSpec((1, _NHID), lambda i: (0, 0)),
            pl.BlockSpec((1, 1), lambda i: (0, 0)),
        ],
        out_specs=pl.BlockSpec((_NSUP, _TSUP), lambda i: (0, 0)),
        out_shape=jax.ShapeDtypeStruct((_NSUP, _TSUP), jnp.float32),
        scratch_shapes=[pltpu.VMEM((_N, _FEAT), jnp.float32)],
        compiler_params=pltpu.CompilerParams(
            dimension_semantics=("arbitrary",),
        ),
    )(adj, x, wt, wb, b.reshape(1, _NHID), W1, b1.reshape(1, _NHID),
      alpha.reshape(1, _NHID), W2.reshape(1, _NHID), b2.reshape(1, 1))
    return out.reshape(-1)


# speedup vs baseline: 1.0233x; 1.0179x over previous
import jax
import jax.numpy as jnp
from jax.experimental import pallas as pl
from jax.experimental.pallas import tpu as pltpu

_N = 10000
_FEAT = 128
_NHID = 256

_HALF = _N // 2
_TMC = 200
_NCH = _HALF // _TMC
_NBUF = 2
_TSUP = _TMC
_NSUP = _N // _TSUP


def _outer(adj_hbm, x_ref, wt_ref, wb_ref, b_ref, w1_ref, b1_ref,
           alpha_ref, w2_ref, b2_ref, out_ref, agg_ref):

    def _mlp_tail(s, orow):
        xm = x_ref[pl.ds(s, _TSUP), :]
        agg = agg_ref[pl.ds(s, _TSUP), :]
        h = jnp.dot(xm, wt_ref[...], preferred_element_type=jnp.float32)
        h += jnp.dot(agg, wb_ref[...], preferred_element_type=jnp.float32)
        h = jnp.maximum(h + b_ref[...], 0.0)
        h1 = jnp.dot(h, w1_ref[...], preferred_element_type=jnp.float32)
        h1 += b1_ref[...]
        h1 = jnp.where(h1 >= 0, h1, alpha_ref[...] * h1)
        pred = jnp.sum(h1 * w2_ref[...], axis=1) + b2_ref[0, 0]
        out_ref[orow, :] = pred

    def _chunk(adj_blk0):
        i = pl.program_id(0)
        agg_ref[pl.ds(i * _TMC, _TMC), :] = jnp.dot(
            adj_blk0[...], x_ref[...], preferred_element_type=jnp.float32)
        _mlp_tail(i * _TMC, i)

    pipe = pltpu.emit_pipeline(
        _chunk,
        grid=(2 * _NCH,),
        in_specs=[
            pl.BlockSpec((_TMC, _N), lambda i: (i, 0),
                         pipeline_mode=pl.Buffered(buffer_count=3)),
        ],
    )
    pipe(adj_hbm)


def kernel(x, adj, W, b, W1, b1, alpha, W2, b2):
    wt = W[:_FEAT]
    wb = W[_FEAT:]
    out = pl.pallas_call(
        _outer,
        grid=(1,),
        in_specs=[
            pl.BlockSpec(memory_space=pltpu.MemorySpace.HBM),
            pl.BlockSpec((_N, _FEAT), lambda i: (0, 0)),
            pl.BlockSpec((_FEAT, _NHID), lambda i: (0, 0)),
            pl.BlockSpec((_FEAT, _NHID), lambda i: (0, 0)),
            pl.BlockSpec((1, _NHID), lambda i: (0, 0)),
            pl.BlockSpec((_NHID, _NHID), lambda i: (0, 0)),
            pl.BlockSpec((1, _NHID), lambda i: (0, 0)),
            pl.BlockSpec((1, _NHID), lambda i: (0, 0)),
            pl.BlockSpec((1, _NHID), lambda i: (0, 0)),
            pl.BlockSpec((1, 1), lambda i: (0, 0)),
        ],
        out_specs=pl.BlockSpec((_NSUP, _TSUP), lambda i: (0, 0)),
        out_shape=jax.ShapeDtypeStruct((_NSUP, _TSUP), jnp.float32),
        scratch_shapes=[pltpu.VMEM((_N, _FEAT), jnp.float32)],
        compiler_params=pltpu.CompilerParams(
            dimension_semantics=("arbitrary",),
        ),
    )(adj, x, wt, wb, b.reshape(1, _NHID), W1, b1.reshape(1, _NHID),
      alpha.reshape(1, _NHID), W2.reshape(1, _NHID), b2.reshape(1, 1))
    return out.reshape(-1)
